# Initial kernel scaffold; baseline (speedup 1.0000x reference)
import jax
import jax.numpy as jnp
from jax.experimental import pallas as pl


def kernel(mlvl_feats, lidar2img, pred_boxes, pred_logits, box_feats, view3d, view5d, query_weight, mask_embedding):
    raise NotImplementedError("write your pallas kernel here")


# speedup vs baseline: 3.7266x; 3.7266x over previous
import jax
import jax.numpy as jnp
from jax import lax
from jax.experimental import pallas as pl
from jax.experimental.pallas import tpu as pltpu

_H = 100
_W = 100
_D = 256
_NA = 100
_NP = 900
_NQ = 900
_NC = 11


def _body(sc_ref, scr_ref, cpl_ref, cqk_ref, qwt_ref, me_ref, out_ref,
          kept_scr):
    j = pl.program_id(1)

    @pl.when(j == 0)
    def _phase_a():
        sc = sc_ref[0]
        scr = scr_ref[0]
        ii = lax.broadcasted_iota(jnp.int32, (_NQ, _NQ), 0)
        jj = lax.broadcasted_iota(jnp.int32, (_NQ, _NQ), 1)
        gt = scr > sc
        eq = (scr == sc) & (jj < ii)
        r = jnp.sum((gt | eq).astype(jnp.float32), axis=1, keepdims=True)
        sel = r < float(_NA)
        qf = ((r == 2.0) | (r == 3.0)).astype(jnp.int32)
        c = qf * cqk_ref[0] + (1 - qf) * cpl_ref[0]
        x0i = c[:, 0:1]
        y0i = c[:, 1:2]
        x1i = c[:, 2:3]
        y1i = c[:, 3:4]

        ax = lax.broadcasted_iota(jnp.int32, (_NQ, _W), 1)
        inx = ((ax >= x0i) & (ax <= x1i) & sel).astype(jnp.float32)
        iny = ((ax >= y0i) & (ax <= y1i)).astype(jnp.float32)
        counts = lax.dot_general(iny, inx, (((0,), (0,)), ((), ())),
                                 preferred_element_type=jnp.float32)
        mf = (counts > 0.0).astype(jnp.float32)

        gi = lax.broadcasted_iota(jnp.int32, (_H, _W), 0)
        gj = lax.broadcasted_iota(jnp.int32, (_H, _W), 1)
        tri = (gi < gj).astype(jnp.float32)
        within = lax.dot_general(mf, tri, (((1,), (0,)), ((), ())),
                                 preferred_element_type=jnp.float32)
        rowtot = jnp.sum(mf, axis=1, keepdims=True)
        tril = (gj < gi).astype(jnp.float32)
        rowexcl = lax.dot_general(tril, rowtot, (((1,), (0,)), ((), ())),
                                  preferred_element_type=jnp.float32)
        arank = rowexcl + within
        total = jnp.sum(jnp.sum(mf, axis=1, keepdims=True), axis=0,
                        keepdims=True)
        hw = (gi * _W + gj).astype(jnp.float32)
        keep_act = (arank < float(_NP)).astype(jnp.float32)
        keep_ina = (total + (hw - arank) < float(_NP)).astype(jnp.float32)
        kept_scr[...] = mf * keep_act + (1.0 - mf) * keep_ina

    kr = kept_scr[...][None]
    out_ref[0] = kr * qwt_ref[...] + (1.0 - kr) * me_ref[...]


def _box_xyxy(b):
    cx, cy, w, h = b[..., 0], b[..., 1], b[..., 2], b[..., 3]
    return jnp.stack([cx - 0.5 * w, cy - 0.5 * h, cx + 0.5 * w, cy + 0.5 * h],
                     axis=-1)


def kernel(mlvl_feats, lidar2img, pred_boxes, pred_logits, box_feats,
           view3d, view5d, query_weight, mask_embedding):
    del mlvl_feats, lidar2img, box_feats, view3d
    b = pred_boxes.shape[0]

    scores = jnp.max(jax.nn.softmax(pred_logits, axis=-1)[..., :-1], axis=-1)
    pb = pred_boxes[..., :4]
    pb_plain = pb.at[..., 2:4].set(jnp.exp(pb[..., 2:4]))
    pb_quirk = jnp.exp(pb_plain)

    def coords_of(pbv):
        cc = jnp.pad(_box_xyxy(pbv), ((0, 0), (0, 0), (0, 1)),
                     constant_values=1.0)
        return jnp.einsum('bij,bNj->bNi', view5d, cc)[..., :4].astype(jnp.int32)

    cpl = coords_of(pb_plain)
    cqk = coords_of(pb_quirk)

    dc = 32
    qwt = query_weight.T.reshape(_D, _H, _W)
    me = mask_embedding.reshape(_D, 1, 1)
    out = pl.pallas_call(
        _body,
        grid=(b, _D // dc),
        in_specs=[
            pl.BlockSpec((1, _NQ, 1), lambda bi, ji: (bi, 0, 0)),
            pl.BlockSpec((1, 1, _NQ), lambda bi, ji: (bi, 0, 0)),
            pl.BlockSpec((1, _NQ, 4), lambda bi, ji: (bi, 0, 0)),
            pl.BlockSpec((1, _NQ, 4), lambda bi, ji: (bi, 0, 0)),
            pl.BlockSpec((dc, _H, _W), lambda bi, ji: (ji, 0, 0)),
            pl.BlockSpec((dc, 1, 1), lambda bi, ji: (ji, 0, 0)),
        ],
        out_specs=pl.BlockSpec((1, dc, _H, _W), lambda bi, ji: (bi, ji, 0, 0)),
        out_shape=jax.ShapeDtypeStruct((b, _D, _H, _W), jnp.float32),
        scratch_shapes=[pltpu.VMEM((_H, _W), jnp.float32)],
    )(scores.reshape(b, _NQ, 1), scores.reshape(b, 1, _NQ), cpl, cqk, qwt, me)
    return out


# speedup vs baseline: 3.8711x; 1.0388x over previous
import jax
import jax.numpy as jnp
from jax import lax
from jax.experimental import pallas as pl
from jax.experimental.pallas import tpu as pltpu

_H = 100
_W = 100
_D = 256
_NA = 100
_NP = 900
_NQ = 900
_DC = 32


def _mask_body(sc_ref, scr_ref, cpl_ref, cqk_ref, kept_ref):
    sc = sc_ref[0]
    scr = scr_ref[0]
    ii = lax.broadcasted_iota(jnp.int32, (_NQ, _NQ), 0)
    jj = lax.broadcasted_iota(jnp.int32, (_NQ, _NQ), 1)
    gt = scr > sc
    eq = (scr == sc) & (jj < ii)
    r = jnp.sum((gt | eq).astype(jnp.float32), axis=1, keepdims=True)
    sel = r < float(_NA)
    qf = ((r == 2.0) | (r == 3.0)).astype(jnp.int32)
    c = qf * cqk_ref[0] + (1 - qf) * cpl_ref[0]
    x0i = c[:, 0:1]
    y0i = c[:, 1:2]
    x1i = c[:, 2:3]
    y1i = c[:, 3:4]

    ax = lax.broadcasted_iota(jnp.int32, (_NQ, _W), 1)
    inx = ((ax >= x0i) & (ax <= x1i) & sel).astype(jnp.float32)
    iny = ((ax >= y0i) & (ax <= y1i)).astype(jnp.float32)
    counts = lax.dot_general(iny, inx, (((0,), (0,)), ((), ())),
                             preferred_element_type=jnp.float32)
    mf = (counts > 0.0).astype(jnp.float32)

    gi = lax.broadcasted_iota(jnp.int32, (_H, _W), 0)
    gj = lax.broadcasted_iota(jnp.int32, (_H, _W), 1)
    tri = (gi < gj).astype(jnp.float32)
    within = lax.dot_general(mf, tri, (((1,), (0,)), ((), ())),
                             preferred_element_type=jnp.float32)
    rowtot = jnp.sum(mf, axis=1, keepdims=True)
    tril = (gj < gi).astype(jnp.float32)
    rowexcl = lax.dot_general(tril, rowtot, (((1,), (0,)), ((), ())),
                              preferred_element_type=jnp.float32)
    arank = rowexcl + within
    total = jnp.sum(jnp.sum(mf, axis=1, keepdims=True), axis=0,
                    keepdims=True)
    hw = (gi * _W + gj).astype(jnp.float32)
    keep_act = (arank < float(_NP)).astype(jnp.float32)
    keep_ina = (total + (hw - arank) < float(_NP)).astype(jnp.float32)
    kept_ref[0] = mf * keep_act + (1.0 - mf) * keep_ina


def _stream_body(kr_ref, qwt_ref, me_ref, out_ref):
    kr = kr_ref[0]
    out_ref[0] = kr * qwt_ref[...] + (1.0 - kr) * me_ref[...]


def _box_xyxy(b):
    cx, cy, w, h = b[..., 0], b[..., 1], b[..., 2], b[..., 3]
    return jnp.stack([cx - 0.5 * w, cy - 0.5 * h, cx + 0.5 * w, cy + 0.5 * h],
                     axis=-1)


def kernel(mlvl_feats, lidar2img, pred_boxes, pred_logits, box_feats,
           view3d, view5d, query_weight, mask_embedding):
    del mlvl_feats, lidar2img, box_feats, view3d
    b = pred_boxes.shape[0]

    scores = jnp.max(jax.nn.softmax(pred_logits, axis=-1)[..., :-1], axis=-1)
    pb = pred_boxes[..., :4]
    pb_plain = pb.at[..., 2:4].set(jnp.exp(pb[..., 2:4]))
    pb_quirk = jnp.exp(pb_plain)

    def coords_of(pbv):
        cc = jnp.pad(_box_xyxy(pbv), ((0, 0), (0, 0), (0, 1)),
                     constant_values=1.0)
        return jnp.einsum('bij,bNj->bNi', view5d, cc)[..., :4].astype(jnp.int32)

    cpl = coords_of(pb_plain)
    cqk = coords_of(pb_quirk)

    kept = pl.pallas_call(
        _mask_body,
        grid=(b,),
        in_specs=[
            pl.BlockSpec((1, _NQ, 1), lambda bi: (bi, 0, 0)),
            pl.BlockSpec((1, 1, _NQ), lambda bi: (bi, 0, 0)),
            pl.BlockSpec((1, _NQ, 4), lambda bi: (bi, 0, 0)),
            pl.BlockSpec((1, _NQ, 4), lambda bi: (bi, 0, 0)),
        ],
        out_specs=pl.BlockSpec((1, _H, _W), lambda bi: (bi, 0, 0)),
        out_shape=jax.ShapeDtypeStruct((b, _H, _W), jnp.float32),
    )(scores.reshape(b, _NQ, 1), scores.reshape(b, 1, _NQ), cpl, cqk)

    keptf = kept.reshape(b, 1, _H * _W)
    qwt = query_weight.T
    me = mask_embedding.reshape(_D, 1)
    out = pl.pallas_call(
        _stream_body,
        grid=(_D // _DC, b),
        in_specs=[
            pl.BlockSpec((1, 1, _H * _W), lambda ji, bi: (bi, 0, 0)),
            pl.BlockSpec((_DC, _H * _W), lambda ji, bi: (ji, 0)),
            pl.BlockSpec((_DC, 1), lambda ji, bi: (ji, 0)),
        ],
        out_specs=pl.BlockSpec((1, _DC, _H * _W), lambda ji, bi: (bi, ji, 0)),
        out_shape=jax.ShapeDtypeStruct((b, _D, _H * _W), jnp.float32),
    )(keptf, qwt, me)
    return out.reshape(b, _D, _H, _W)
